# baseline (speedup 1.0000x reference)
import functools

import jax
import jax.numpy as jnp
from jax import lax
from jax.experimental import pallas as pl
from jax.experimental.pallas import tpu as pltpu
from jax.experimental.pallas import tpu_sc as plsc

B = 16384
D = 256
NC, NS = 2, 16
NW = NC * NS
IDX_MINOR = 128
ROWS_PER_W = B // NW
CHUNKS = ROWS_PER_W // IDX_MINOR


def _sc_gather(idx2, table):
    mesh = plsc.VectorSubcoreMesh(
        core_axis_name="c", subcore_axis_name="s",
        num_cores=NC, num_subcores=NS)

    @functools.partial(
        pl.kernel,
        mesh=mesh,
        out_type=jax.ShapeDtypeStruct((B, D), jnp.float32),
        scratch_types=[
            pltpu.VMEM((CHUNKS, IDX_MINOR), jnp.int32),
            pltpu.VMEM((IDX_MINOR, D), jnp.float32),
            pltpu.VMEM((IDX_MINOR, D), jnp.float32),
            pltpu.SemaphoreType.DMA,
            pltpu.SemaphoreType.DMA,
            pltpu.SemaphoreType.DMA,
        ],
    )
    def gather_k(idx_hbm, table_hbm, out_hbm, idx_v, buf0, buf1,
                 gsem, osem0, osem1):
        wid = lax.axis_index("s") * NC + lax.axis_index("c")
        pltpu.sync_copy(idx_hbm.at[pl.ds(wid * CHUNKS, CHUNKS)], idx_v)
        bufs = (buf0, buf1)
        osems = (osem0, osem1)
        out_copies = [None, None]
        for j in range(CHUNKS):
            k = j % 2
            if out_copies[k] is not None:
                out_copies[k].wait()
            pltpu.async_copy(table_hbm.at[idx_v.at[j]], bufs[k], gsem).wait()
            dst = out_hbm.at[pl.ds(wid * ROWS_PER_W + j * IDX_MINOR,
                                   IDX_MINOR)]
            out_copies[k] = pltpu.async_copy(bufs[k], dst, osems[k])
        for c in out_copies:
            if c is not None:
                c.wait()

    return gather_k(idx2, table)


def _dense_body(x_ref, v4_ref, w1_ref, e2_ref, e3_ref, w5_ref, b5_ref,
                w6_ref, b6_ref, y_ref):
    x = x_ref[...]
    v1 = jnp.tanh(jnp.dot(x, w1_ref[...],
                          preferred_element_type=jnp.float32))
    cols = lax.broadcasted_iota(jnp.int32, (1, 8), 1).astype(jnp.float32)
    oh3 = (x[:, 3:4] == cols).astype(jnp.float32)
    oh4 = (x[:, 4:5] == cols).astype(jnp.float32)
    v2 = jnp.dot(oh3, e2_ref[...], preferred_element_type=jnp.float32)
    v3 = jnp.dot(oh4, e3_ref[...], preferred_element_type=jnp.float32)
    h = (jnp.dot(v1, w5_ref[0:8, :], preferred_element_type=jnp.float32)
         + jnp.dot(v2, w5_ref[8:16, :], preferred_element_type=jnp.float32)
         + jnp.dot(v3, w5_ref[16:24, :], preferred_element_type=jnp.float32)
         + b5_ref[...])
    v5 = jnp.tanh(h)
    y = (jnp.dot(v4_ref[...], w6_ref[0:D, :],
                 preferred_element_type=jnp.float32)
         + jnp.dot(v5, w6_ref[D:2 * D, :], preferred_element_type=jnp.float32)
         + b6_ref[...])
    y_ref[...] = jnp.tanh(y)


def kernel(p1, p2, p5, p3, p4, p6, W1, b1, E2, E3, E4, W5, b5, W6, b6):
    f32 = jnp.float32
    X = jnp.concatenate(
        [p1, p2, p5,
         p3[:, None].astype(f32), p4[:, None].astype(f32),
         jnp.zeros((B, 3), f32)], axis=1)
    X = X.at[:, 5].set(1.0)
    W1p = jnp.zeros((8, 8), f32)
    W1p = W1p.at[0:3, :].set(W1)
    W1p = W1p.at[5, :].set(b1)
    E2p = jnp.zeros((8, 8), f32).at[0:E2.shape[0], :].set(E2)
    E3p = jnp.zeros((8, 8), f32).at[0:E3.shape[0], :].set(E3)

    idx2 = p6.astype(jnp.int32).reshape(B // IDX_MINOR, IDX_MINOR)
    v4 = lax.dynamic_slice(E4, (0, 0), (B, D))

    BK = 2048
    grid = (B // BK,)
    rep = lambda i: (0, 0)
    y = pl.pallas_call(
        _dense_body,
        grid=grid,
        in_specs=[
            pl.BlockSpec((BK, 8), lambda i: (i, 0)),
            pl.BlockSpec((BK, D), lambda i: (i, 0)),
            pl.BlockSpec((8, 8), rep),
            pl.BlockSpec((8, 8), rep),
            pl.BlockSpec((8, 8), rep),
            pl.BlockSpec((24, D), rep),
            pl.BlockSpec((1, D), rep),
            pl.BlockSpec((2 * D, 128), rep),
            pl.BlockSpec((1, 128), rep),
        ],
        out_specs=pl.BlockSpec((BK, 128), lambda i: (i, 0)),
        out_shape=jax.ShapeDtypeStruct((B, 128), f32),
    )(X, v4, W1p, E2p, E3p, W5, b5[None, :], W6, b6[None, :])
    return y


# speedup vs baseline: 1.2533x; 1.2533x over previous
import functools

import jax
import jax.numpy as jnp
from jax import lax
from jax.experimental import pallas as pl
from jax.experimental.pallas import tpu as pltpu
from jax.experimental.pallas import tpu_sc as plsc

B = 16384
D = 256
NC, NS = 2, 16
NW = NC * NS
IDX_MINOR = 128
ROWS_PER_W = B // NW
CHUNKS = ROWS_PER_W // IDX_MINOR
NBUF = 3


def _sc_gather(idx2, table):
    mesh = plsc.VectorSubcoreMesh(
        core_axis_name="c", subcore_axis_name="s",
        num_cores=NC, num_subcores=NS)

    @functools.partial(
        pl.kernel,
        mesh=mesh,
        out_type=jax.ShapeDtypeStruct((B, D), jnp.float32),
        scratch_types=(
            [pltpu.VMEM((CHUNKS, IDX_MINOR), jnp.int32)]
            + [pltpu.VMEM((IDX_MINOR, D), jnp.float32) for _ in range(NBUF)]
            + [pltpu.SemaphoreType.DMA for _ in range(2 * NBUF)]
        ),
    )
    def gather_k(idx_hbm, table_hbm, out_hbm, idx_v, *scratch):
        bufs = scratch[:NBUF]
        gsems = scratch[NBUF:2 * NBUF]
        osems = scratch[2 * NBUF:]
        wid = lax.axis_index("s") * NC + lax.axis_index("c")
        pltpu.sync_copy(idx_hbm.at[pl.ds(wid * CHUNKS, CHUNKS)], idx_v)
        gathers = [None] * CHUNKS
        outs = [None] * CHUNKS

        def fire_gather(j):
            k = j % NBUF
            gathers[j] = pltpu.async_copy(
                table_hbm.at[idx_v.at[j]], bufs[k], gsems[k])

        def fire_out(j):
            k = j % NBUF
            dst = out_hbm.at[pl.ds(wid * ROWS_PER_W + j * IDX_MINOR,
                                   IDX_MINOR)]
            outs[j] = pltpu.async_copy(bufs[k], dst, osems[k])

        for j in range(min(NBUF, CHUNKS)):
            fire_gather(j)
        for j in range(CHUNKS):
            gathers[j].wait()
            fire_out(j)
            nxt = j + NBUF
            if nxt < CHUNKS:
                outs[nxt - NBUF].wait()
                fire_gather(nxt)
        for j in range(max(0, CHUNKS - NBUF), CHUNKS):
            outs[j].wait()

    return gather_k(idx2, table)


def _dense_body(p1_ref, p2_ref, p5_ref, p3_ref, p4_ref, v4_ref,
                w1_ref, b1_ref, wc_ref, b5_ref, w6_ref, b6_ref, y_ref):
    f32 = jnp.float32
    v1 = jnp.tanh(p1_ref[...] * w1_ref[0:1, :]
                  + p2_ref[...] * w1_ref[1:2, :]
                  + p5_ref[...] * w1_ref[2:3, :]
                  + b1_ref[...])
    cols = lax.broadcasted_iota(jnp.int32, (1, 8), 1)
    p3c = p3_ref[...][:, None]
    p4c = p4_ref[...][:, None]
    oh3 = (p3c == cols).astype(f32)
    oh4 = (p4c == cols).astype(f32)
    u = jnp.concatenate([v1, oh3, oh4], axis=1)
    v5 = jnp.tanh(jnp.dot(u, wc_ref[...],
                          preferred_element_type=f32) + b5_ref[...])
    y = (jnp.dot(v4_ref[...], w6_ref[0:D, :], preferred_element_type=f32)
         + jnp.dot(v5, w6_ref[D:2 * D, :], preferred_element_type=f32)
         + b6_ref[...])
    y_ref[...] = jnp.tanh(y)


def kernel(p1, p2, p5, p3, p4, p6, W1, b1, E2, E3, E4, W5, b5, W6, b6):
    f32 = jnp.float32
    W1p = jnp.zeros((8, 8), f32).at[0:3, :].set(W1)
    Wc = jnp.zeros((24, D), f32)
    Wc = Wc.at[0:8, :].set(W5[0:8, :])
    Wc = Wc.at[8:8 + E2.shape[0], :].set(E2 @ W5[8:16, :])
    Wc = Wc.at[16:16 + E3.shape[0], :].set(E3 @ W5[16:24, :])

    p3i = p3.astype(jnp.int32)
    p4i = p4.astype(jnp.int32)
    idx2 = p6.astype(jnp.int32).reshape(B // IDX_MINOR, IDX_MINOR)
    v4 = _sc_gather(idx2, E4)

    BK = 2048
    grid = (B // BK,)
    rep = lambda i: (0, 0)
    y = pl.pallas_call(
        _dense_body,
        grid=grid,
        in_specs=[
            pl.BlockSpec((BK, 1), lambda i: (i, 0)),
            pl.BlockSpec((BK, 1), lambda i: (i, 0)),
            pl.BlockSpec((BK, 1), lambda i: (i, 0)),
            pl.BlockSpec((BK,), lambda i: (i,)),
            pl.BlockSpec((BK,), lambda i: (i,)),
            pl.BlockSpec((BK, D), lambda i: (i, 0)),
            pl.BlockSpec((8, 8), rep),
            pl.BlockSpec((1, 8), rep),
            pl.BlockSpec((24, D), rep),
            pl.BlockSpec((1, D), rep),
            pl.BlockSpec((2 * D, 128), rep),
            pl.BlockSpec((1, 128), rep),
        ],
        out_specs=pl.BlockSpec((BK, 128), lambda i: (i, 0)),
        out_shape=jax.ShapeDtypeStruct((B, 128), f32),
    )(p1, p2, p5, p3i, p4i, v4, W1p, b1[None, :], Wc, b5[None, :],
      W6, b6[None, :])
    return y


# speedup vs baseline: 1.6922x; 1.3502x over previous
import functools

import jax
import jax.numpy as jnp
from jax import lax
from jax.experimental import pallas as pl
from jax.experimental.pallas import tpu as pltpu
from jax.experimental.pallas import tpu_sc as plsc

B = 16384
D = 256
NC, NS = 2, 16
NW = NC * NS
IDX_MINOR = 128
ROWS_PER_W = B // NW
CHUNKS = ROWS_PER_W // IDX_MINOR
NBUF = 3


def _sc_gather(idx2, table):
    mesh = plsc.VectorSubcoreMesh(
        core_axis_name="c", subcore_axis_name="s",
        num_cores=NC, num_subcores=NS)

    @functools.partial(
        pl.kernel,
        mesh=mesh,
        out_type=jax.ShapeDtypeStruct((B, D), jnp.float32),
        scratch_types=(
            [pltpu.VMEM((CHUNKS, IDX_MINOR), jnp.int32)]
            + [pltpu.VMEM((IDX_MINOR, D), jnp.float32) for _ in range(NBUF)]
            + [pltpu.SemaphoreType.DMA for _ in range(2 * NBUF)]
        ),
    )
    def gather_k(idx_hbm, table_hbm, out_hbm, idx_v, *scratch):
        bufs = scratch[:NBUF]
        gsems = scratch[NBUF:2 * NBUF]
        osems = scratch[2 * NBUF:]
        wid = lax.axis_index("s") * NC + lax.axis_index("c")
        pltpu.sync_copy(idx_hbm.at[pl.ds(wid * CHUNKS, CHUNKS)], idx_v)
        gathers = [None] * CHUNKS
        outs = [None] * CHUNKS

        def fire_gather(j):
            k = j % NBUF
            gathers[j] = pltpu.async_copy(
                table_hbm.at[idx_v.at[j]], bufs[k], gsems[k])

        def fire_out(j):
            k = j % NBUF
            dst = out_hbm.at[pl.ds(wid * ROWS_PER_W + j * IDX_MINOR,
                                   IDX_MINOR)]
            outs[j] = pltpu.async_copy(bufs[k], dst, osems[k])

        for j in range(min(NBUF, CHUNKS)):
            fire_gather(j)
        for j in range(CHUNKS):
            gathers[j].wait()
            fire_out(j)
            nxt = j + NBUF
            if nxt < CHUNKS:
                outs[nxt - NBUF].wait()
                fire_gather(nxt)
        for j in range(max(0, CHUNKS - NBUF), CHUNKS):
            outs[j].wait()

    return gather_k(idx2, table)


def _dense_body(p_ref, v4_ref, w1_ref, wct_ref, w6_ref, b6_ref, y_ref):
    f32 = jnp.float32
    pb = p_ref[...]
    v1t = jnp.tanh(jnp.dot(w1_ref[...], pb, preferred_element_type=f32))
    rows = lax.broadcasted_iota(jnp.int32, (8, 1), 0).astype(f32)
    oh3t = (pb[3:4, :] == rows).astype(f32)
    oh4t = (pb[4:5, :] == rows).astype(f32)
    ones = pb[5:6, :]
    ut = jnp.concatenate([v1t, oh3t, oh4t[0:7, :], ones], axis=0)
    v5t = jnp.tanh(jnp.dot(wct_ref[...], ut, preferred_element_type=f32))
    y = (jnp.dot(v4_ref[...], w6_ref[0:D, :], preferred_element_type=f32)
         + lax.dot_general(v5t, w6_ref[D:2 * D, :],
                           (((0,), (0,)), ((), ())),
                           preferred_element_type=f32)
         + b6_ref[...])
    y_ref[...] = jnp.tanh(y)


def kernel(p1, p2, p5, p3, p4, p6, W1, b1, E2, E3, E4, W5, b5, W6, b6):
    f32 = jnp.float32
    W1e = jnp.zeros((8, 8), f32)
    W1e = W1e.at[:, 0:3].set(W1.T)
    W1e = W1e.at[:, 5].set(b1)
    WcT = jnp.zeros((D, 24), f32)
    WcT = WcT.at[:, 0:8].set(W5[0:8, :].T)
    WcT = WcT.at[:, 8:8 + E2.shape[0]].set((E2 @ W5[8:16, :]).T)
    WcT = WcT.at[:, 16:16 + E3.shape[0]].set((E3 @ W5[16:24, :]).T)
    WcT = WcT.at[:, 23].set(b5)

    P = jnp.concatenate([
        p1.T, p2.T, p5.T,
        p3.astype(f32)[None, :], p4.astype(f32)[None, :],
        jnp.ones((1, B), f32), jnp.zeros((2, B), f32)], axis=0)

    idx2 = p6.astype(jnp.int32).reshape(B // IDX_MINOR, IDX_MINOR)
    v4 = _sc_gather(idx2, E4)

    BK = 2048
    grid = (B // BK,)
    rep = lambda i: (0, 0)
    y = pl.pallas_call(
        _dense_body,
        grid=grid,
        in_specs=[
            pl.BlockSpec((8, BK), lambda i: (0, i)),
            pl.BlockSpec((BK, D), lambda i: (i, 0)),
            pl.BlockSpec((8, 8), rep),
            pl.BlockSpec((D, 24), rep),
            pl.BlockSpec((2 * D, 128), rep),
            pl.BlockSpec((1, 128), rep),
        ],
        out_specs=pl.BlockSpec((BK, 128), lambda i: (i, 0)),
        out_shape=jax.ShapeDtypeStruct((B, 128), f32),
    )(P, v4, W1e, WcT, W6, b6[None, :])
    return y
